# baseline (speedup 1.0000x reference)
import functools

import jax
import jax.numpy as jnp
from jax import lax
from jax.experimental import pallas as pl
from jax.experimental.pallas import tpu as pltpu
from jax.experimental.pallas import tpu_sc as plsc

_NB = 5000
_C = 128
_FEAT = (256, 128, 64, 32)
_OUT_HW = 7
_NBINS = _OUT_HW * _OUT_HW
_NPTS = _NBINS * 16
_NCHUNK = 7
_CHUNK_ROWS = 112
_NBUF = 4
_NWORKERS = 32
_PER_TILE = 157
_NB_PAD = _NWORKERS * _PER_TILE
_TABLE_ROWS = sum(s * s for s in _FEAT)
_CANONICAL_SCALE = 224.0
_CANONICAL_LEVEL = 4.0


def _splat(ref, pos):
    return plsc.load_gather(ref, [jnp.full((16,), pos, jnp.int32)])


_TBLK = 512


def _transpose_body(s_ref, d_ref, o_ref):
    del d_ref
    o_ref[...] = jnp.transpose(s_ref[...], (1, 0))


def _transpose_body0(s_ref, o_ref):
    o_ref[...] = jnp.transpose(s_ref[...], (1, 0))


def _build_table(feats):
    table = None
    off = 0
    for f in feats:
        hw = f.shape[1] * f.shape[2]
        src = f.reshape(_C, hw)
        nblk = hw // _TBLK
        off_blk = off // _TBLK
        out_shape = jax.ShapeDtypeStruct((_TABLE_ROWS, _C), jnp.float32)
        out_spec = pl.BlockSpec(
            (_TBLK, _C), lambda g, o=off_blk: (o + g, 0))
        in_spec = pl.BlockSpec((_C, _TBLK), lambda g: (0, g))
        if table is None:
            table = pl.pallas_call(
                _transpose_body0,
                grid=(nblk,),
                in_specs=[in_spec],
                out_specs=out_spec,
                out_shape=out_shape,
            )(src)
        else:
            table = pl.pallas_call(
                _transpose_body,
                grid=(nblk,),
                in_specs=[in_spec,
                          pl.BlockSpec(memory_space=pl.ANY)],
                out_specs=out_spec,
                out_shape=out_shape,
                input_output_aliases={1: 0},
            )(src, table)
        off += hw
    return table


def _roi_body(table, bparams, lvlp, out, bp_v, lvlp_v, idx_v, w_v, rows_v,
              acc_v, sem_stage, g0, g1, g2, g3):
    gsems = (g0, g1, g2, g3)
    cid = lax.axis_index("c")
    sid = lax.axis_index("s")
    wid = sid * 2 + cid
    base = wid * _PER_TILE
    nb = jnp.minimum(_PER_TILE, _NB - base)

    pltpu.async_copy(bparams.at[pl.ds(base * 8, _PER_TILE * 8)], bp_v,
                     sem_stage).wait()
    pltpu.async_copy(lvlp, lvlp_v, sem_stage).wait()

    lane = lax.iota(jnp.int32, 16)
    syf = ((lane >> 3) & 1).astype(jnp.float32) + 0.5
    sxf = ((lane >> 2) & 1).astype(jnp.float32) + 0.5
    dym = ((lane >> 1) & 1) == 1
    dxm = (lane & 1) == 1
    lane49 = lane * _NBINS

    def per_box(i, _):
        b8 = i * 8
        x1 = _splat(bp_v, b8 + 0)
        y1 = _splat(bp_v, b8 + 1)
        x2 = _splat(bp_v, b8 + 2)
        y2 = _splat(bp_v, b8 + 3)
        li = _splat(bp_v, b8 + 4).astype(jnp.int32) * 4
        scale = plsc.load_gather(lvlp_v, [li])
        wf = plsc.load_gather(lvlp_v, [li + 1])
        hf = plsc.load_gather(lvlp_v, [li + 2])
        off_i = plsc.load_gather(lvlp_v, [li + 3]).astype(jnp.int32)
        rx1 = x1 * scale
        ry1 = y1 * scale
        roi_w = jnp.maximum(x2 * scale - rx1, 1.0)
        roi_h = jnp.maximum(y2 * scale - ry1, 1.0)
        bin_w = roi_w / 7.0
        bin_h = roi_h / 7.0
        bin_w2 = bin_w * 0.5
        bin_h2 = bin_h * 0.5
        wm1 = wf - 1.0
        hm1 = hf - 1.0
        w_i = wf.astype(jnp.int32)
        wm1_i = wm1.astype(jnp.int32)
        hm1_i = hm1.astype(jnp.int32)

        def per_vec(v, _):
            oh = v // _OUT_HW
            ow = v - oh * _OUT_HW
            ys = (ry1 + oh.astype(jnp.float32) * bin_h) + syf * bin_h2
            xs = (rx1 + ow.astype(jnp.float32) * bin_w) + sxf * bin_w2
            valid = ((ys >= -1.0) & (ys <= hf)) & ((xs >= -1.0) & (xs <= wf))
            ycl = jnp.minimum(jnp.maximum(ys, 0.0), hm1)
            xcl = jnp.minimum(jnp.maximum(xs, 0.0), wm1)
            y0i = ycl.astype(jnp.int32)
            x0i = xcl.astype(jnp.int32)
            ly = ycl - y0i.astype(jnp.float32)
            lx = xcl - x0i.astype(jnp.float32)
            wy = jnp.where(dym, ly, 1.0 - ly)
            wx = jnp.where(dxm, lx, 1.0 - lx)
            yc = jnp.where(dym, jnp.minimum(y0i + 1, hm1_i), y0i)
            xc = jnp.where(dxm, jnp.minimum(x0i + 1, wm1_i), x0i)
            w = (wy * wx) * jnp.where(valid, 0.25, 0.0)
            idx = (off_i + yc * w_i) + xc
            r = v // _OUT_HW
            cc = (v - r * _OUT_HW) * 16
            idx_v[r, pl.ds(cc, 16)] = idx
            w_v[pl.ds(v * 16, 16)] = w
            return ()

        lax.fori_loop(0, _NBINS, per_vec, (), unroll=False)

        handles = [None] * _NCHUNK

        def fire(j):
            slot = j % _NBUF
            handles[j] = pltpu.async_copy(
                table.at[idx_v.at[j]], rows_v.at[slot], gsems[slot])

        for j in range(_NBUF):
            fire(j)
        for j in range(_NCHUNK):
            slot = j % _NBUF
            handles[j].wait()

            def per_bin(bb, _):
                binno = j * _OUT_HW + bb
                accs = [None] * 8
                rbase = bb * 16
                for l in range(16):
                    wv = _splat(w_v, binno * 16 + l)
                    for c in range(8):
                        row = rows_v[slot, rbase + l, pl.ds(c * 16, 16)]
                        if l == 0:
                            accs[c] = row * wv
                        else:
                            accs[c] = accs[c] + row * wv
                for c in range(8):
                    addr = lane49 + (c * 16 * _NBINS + binno)
                    plsc.store_scatter(acc_v, [addr], accs[c])
                return ()

            lax.fori_loop(0, _OUT_HW, per_bin, (), unroll=False)
            if j + _NBUF < _NCHUNK:
                fire(j + _NBUF)

        pltpu.sync_copy(acc_v, out.at[base + i])
        return ()

    lax.fori_loop(0, nb, per_box, (), unroll=False)


@jax.jit
def _roialign_sc(table, bparams, lvlp):
    mesh = plsc.VectorSubcoreMesh(core_axis_name="c", subcore_axis_name="s")
    f = functools.partial(
        pl.kernel,
        out_type=jax.ShapeDtypeStruct((_NB, _C * _NBINS), jnp.float32),
        mesh=mesh,
        scratch_types=[
            pltpu.VMEM((_PER_TILE * 8,), jnp.float32),
            pltpu.VMEM((16,), jnp.float32),
            pltpu.VMEM((_NCHUNK, _CHUNK_ROWS), jnp.int32),
            pltpu.VMEM((_NPTS,), jnp.float32),
            pltpu.VMEM((_NBUF, _CHUNK_ROWS, _C), jnp.float32),
            pltpu.VMEM((_C * _NBINS,), jnp.float32),
            pltpu.SemaphoreType.DMA,
            pltpu.SemaphoreType.DMA,
            pltpu.SemaphoreType.DMA,
            pltpu.SemaphoreType.DMA,
            pltpu.SemaphoreType.DMA,
        ],
        compiler_params=pltpu.CompilerParams(needs_layout_passes=False),
    )(_roi_body)
    return f(table, bparams, lvlp)


def kernel(feat0, feat1, feat2, feat3, boxes, image_h, image_w):
    feats = [feat0[0], feat1[0], feat2[0], feat3[0]]
    img_w = jnp.asarray(image_w, jnp.float32)
    scales = [jnp.asarray(float(f.shape[-1]), jnp.float32) / img_w
              for f in feats]
    lvl_min = -jnp.log2(scales[0])
    lvl_max = -jnp.log2(scales[-1])
    w = boxes[:, 2] - boxes[:, 0]
    h = boxes[:, 3] - boxes[:, 1]
    s = jnp.sqrt(w * h)
    t = jnp.floor(_CANONICAL_LEVEL + jnp.log2(s / _CANONICAL_SCALE + 1e-6))
    levels = jnp.clip(t, lvl_min, lvl_max) - lvl_min

    table = _build_table(feats)

    offsets = []
    acc = 0
    for sz in _FEAT:
        offsets.append(acc)
        acc += sz * sz
    lvlp = jnp.stack(
        [jnp.stack([scales[l],
                    jnp.asarray(float(_FEAT[l]), jnp.float32),
                    jnp.asarray(float(_FEAT[l]), jnp.float32),
                    jnp.asarray(float(offsets[l]), jnp.float32)])
         for l in range(4)]).reshape(16)

    bparams = jnp.zeros((_NB_PAD, 8), jnp.float32)
    bparams = bparams.at[:_NB, :4].set(boxes)
    bparams = bparams.at[:_NB, 4].set(levels)
    out = _roialign_sc(table, bparams.reshape(-1), lvlp)
    return out.reshape(_NB, _C, _OUT_HW, _OUT_HW)


# speedup vs baseline: 1.0346x; 1.0346x over previous
import functools

import jax
import jax.numpy as jnp
from jax import lax
from jax.experimental import pallas as pl
from jax.experimental.pallas import tpu as pltpu
from jax.experimental.pallas import tpu_sc as plsc

_NB = 5000
_C = 128
_FEAT = (256, 128, 64, 32)
_OUT_HW = 7
_NBINS = _OUT_HW * _OUT_HW
_NPTS = _NBINS * 16
_NCHUNK = 7
_CHUNK_ROWS = 112
_NBUF = 4
_NWORKERS = 32
_PER_TILE = 157
_NB_PAD = _NWORKERS * _PER_TILE
_TABLE_ROWS = sum(s * s for s in _FEAT)
_CANONICAL_SCALE = 224.0
_CANONICAL_LEVEL = 4.0


def _splat(ref, pos):
    return plsc.load_gather(ref, [jnp.full((16,), pos, jnp.int32)])


_HB = 8


def _make_transpose_body(w):
    def body(s_ref, *rest):
        o_ref = rest[-1]
        for r in range(_HB):
            o_ref[pl.ds(r * w, w), :] = jnp.transpose(s_ref[0, :, r, :],
                                                      (1, 0))
    return body


def _build_table(feats):
    table = None
    off = 0
    for f in feats:
        h, w = f.shape[2], f.shape[3]
        rows_per_blk = _HB * w
        off_blk = off // rows_per_blk
        out_shape = jax.ShapeDtypeStruct((_TABLE_ROWS, _C), jnp.float32)
        out_spec = pl.BlockSpec(
            (rows_per_blk, _C), lambda g, o=off_blk: (o + g, 0))
        in_spec = pl.BlockSpec((1, _C, _HB, w), lambda g: (0, 0, g, 0))
        body = _make_transpose_body(w)
        if table is None:
            table = pl.pallas_call(
                body,
                grid=(h // _HB,),
                in_specs=[in_spec],
                out_specs=out_spec,
                out_shape=out_shape,
            )(f)
        else:
            table = pl.pallas_call(
                body,
                grid=(h // _HB,),
                in_specs=[in_spec,
                          pl.BlockSpec(memory_space=pl.ANY)],
                out_specs=out_spec,
                out_shape=out_shape,
                input_output_aliases={1: 0},
            )(f, table)
        off += h * w
    return table


def _roi_body(table, bparams, lvlp, out, bp_v, lvlp_v, idx_v, w_v, rows_v,
              acc_v, sem_stage, g0, g1, g2, g3):
    gsems = (g0, g1, g2, g3)
    cid = lax.axis_index("c")
    sid = lax.axis_index("s")
    wid = sid * 2 + cid
    base = wid * _PER_TILE
    nb = jnp.minimum(_PER_TILE, _NB - base)

    pltpu.async_copy(bparams.at[pl.ds(base * 8, _PER_TILE * 8)], bp_v,
                     sem_stage).wait()
    pltpu.async_copy(lvlp, lvlp_v, sem_stage).wait()

    lane = lax.iota(jnp.int32, 16)
    syf = ((lane >> 3) & 1).astype(jnp.float32) + 0.5
    sxf = ((lane >> 2) & 1).astype(jnp.float32) + 0.5
    dym = ((lane >> 1) & 1) == 1
    dxm = (lane & 1) == 1
    lane49 = lane * _NBINS

    def per_box(i, _):
        b8 = i * 8
        x1 = _splat(bp_v, b8 + 0)
        y1 = _splat(bp_v, b8 + 1)
        x2 = _splat(bp_v, b8 + 2)
        y2 = _splat(bp_v, b8 + 3)
        li = _splat(bp_v, b8 + 4).astype(jnp.int32) * 4
        scale = plsc.load_gather(lvlp_v, [li])
        wf = plsc.load_gather(lvlp_v, [li + 1])
        hf = plsc.load_gather(lvlp_v, [li + 2])
        off_i = plsc.load_gather(lvlp_v, [li + 3]).astype(jnp.int32)
        rx1 = x1 * scale
        ry1 = y1 * scale
        roi_w = jnp.maximum(x2 * scale - rx1, 1.0)
        roi_h = jnp.maximum(y2 * scale - ry1, 1.0)
        bin_w = roi_w / 7.0
        bin_h = roi_h / 7.0
        bin_w2 = bin_w * 0.5
        bin_h2 = bin_h * 0.5
        wm1 = wf - 1.0
        hm1 = hf - 1.0
        w_i = wf.astype(jnp.int32)
        wm1_i = wm1.astype(jnp.int32)
        hm1_i = hm1.astype(jnp.int32)

        def per_vec(v, _):
            oh = v // _OUT_HW
            ow = v - oh * _OUT_HW
            ys = (ry1 + oh.astype(jnp.float32) * bin_h) + syf * bin_h2
            xs = (rx1 + ow.astype(jnp.float32) * bin_w) + sxf * bin_w2
            valid = ((ys >= -1.0) & (ys <= hf)) & ((xs >= -1.0) & (xs <= wf))
            ycl = jnp.minimum(jnp.maximum(ys, 0.0), hm1)
            xcl = jnp.minimum(jnp.maximum(xs, 0.0), wm1)
            y0i = ycl.astype(jnp.int32)
            x0i = xcl.astype(jnp.int32)
            ly = ycl - y0i.astype(jnp.float32)
            lx = xcl - x0i.astype(jnp.float32)
            wy = jnp.where(dym, ly, 1.0 - ly)
            wx = jnp.where(dxm, lx, 1.0 - lx)
            yc = jnp.where(dym, jnp.minimum(y0i + 1, hm1_i), y0i)
            xc = jnp.where(dxm, jnp.minimum(x0i + 1, wm1_i), x0i)
            w = (wy * wx) * jnp.where(valid, 0.25, 0.0)
            idx = (off_i + yc * w_i) + xc
            r = v // _OUT_HW
            cc = (v - r * _OUT_HW) * 16
            idx_v[r, pl.ds(cc, 16)] = idx
            w_v[pl.ds(v * 16, 16)] = w
            return ()

        lax.fori_loop(0, _NBINS, per_vec, (), unroll=False)

        handles = [None] * _NCHUNK

        def fire(j):
            slot = j % _NBUF
            handles[j] = pltpu.async_copy(
                table.at[idx_v.at[j]], rows_v.at[slot], gsems[slot])

        for j in range(_NBUF):
            fire(j)
        for j in range(_NCHUNK):
            slot = j % _NBUF
            handles[j].wait()

            def per_bin(bb, _):
                binno = j * _OUT_HW + bb
                accs = [None] * 8
                rbase = bb * 16
                for l in range(16):
                    wv = _splat(w_v, binno * 16 + l)
                    for c in range(8):
                        row = rows_v[slot, rbase + l, pl.ds(c * 16, 16)]
                        if l == 0:
                            accs[c] = row * wv
                        else:
                            accs[c] = accs[c] + row * wv
                for c in range(8):
                    addr = lane49 + (c * 16 * _NBINS + binno)
                    plsc.store_scatter(acc_v, [addr], accs[c])
                return ()

            lax.fori_loop(0, _OUT_HW, per_bin, (), unroll=False)
            if j + _NBUF < _NCHUNK:
                fire(j + _NBUF)

        pltpu.sync_copy(acc_v, out.at[base + i])
        return ()

    lax.fori_loop(0, nb, per_box, (), unroll=False)


@jax.jit
def _roialign_sc(table, bparams, lvlp):
    mesh = plsc.VectorSubcoreMesh(core_axis_name="c", subcore_axis_name="s")
    f = functools.partial(
        pl.kernel,
        out_type=jax.ShapeDtypeStruct((_NB, _C * _NBINS), jnp.float32),
        mesh=mesh,
        scratch_types=[
            pltpu.VMEM((_PER_TILE * 8,), jnp.float32),
            pltpu.VMEM((16,), jnp.float32),
            pltpu.VMEM((_NCHUNK, _CHUNK_ROWS), jnp.int32),
            pltpu.VMEM((_NPTS,), jnp.float32),
            pltpu.VMEM((_NBUF, _CHUNK_ROWS, _C), jnp.float32),
            pltpu.VMEM((_C * _NBINS,), jnp.float32),
            pltpu.SemaphoreType.DMA,
            pltpu.SemaphoreType.DMA,
            pltpu.SemaphoreType.DMA,
            pltpu.SemaphoreType.DMA,
            pltpu.SemaphoreType.DMA,
        ],
        compiler_params=pltpu.CompilerParams(needs_layout_passes=False),
    )(_roi_body)
    return f(table, bparams, lvlp)


def kernel(feat0, feat1, feat2, feat3, boxes, image_h, image_w):
    feats = [feat0[0], feat1[0], feat2[0], feat3[0]]
    img_w = jnp.asarray(image_w, jnp.float32)
    scales = [jnp.asarray(float(f.shape[-1]), jnp.float32) / img_w
              for f in feats]
    lvl_min = -jnp.log2(scales[0])
    lvl_max = -jnp.log2(scales[-1])
    w = boxes[:, 2] - boxes[:, 0]
    h = boxes[:, 3] - boxes[:, 1]
    s = jnp.sqrt(w * h)
    t = jnp.floor(_CANONICAL_LEVEL + jnp.log2(s / _CANONICAL_SCALE + 1e-6))
    levels = jnp.clip(t, lvl_min, lvl_max) - lvl_min

    table = _build_table([feat0, feat1, feat2, feat3])

    offsets = []
    acc = 0
    for sz in _FEAT:
        offsets.append(acc)
        acc += sz * sz
    lvlp = jnp.stack(
        [jnp.stack([scales[l],
                    jnp.asarray(float(_FEAT[l]), jnp.float32),
                    jnp.asarray(float(_FEAT[l]), jnp.float32),
                    jnp.asarray(float(offsets[l]), jnp.float32)])
         for l in range(4)]).reshape(16)

    bparams = jnp.zeros((_NB_PAD, 8), jnp.float32)
    bparams = bparams.at[:_NB, :4].set(boxes)
    bparams = bparams.at[:_NB, 4].set(levels)
    out = _roialign_sc(table, bparams.reshape(-1), lvlp)
    return out.reshape(_NB, _C, _OUT_HW, _OUT_HW)


# speedup vs baseline: 1.2044x; 1.1641x over previous
import functools

import jax
import jax.numpy as jnp
from jax import lax
from jax.experimental import pallas as pl
from jax.experimental.pallas import tpu as pltpu
from jax.experimental.pallas import tpu_sc as plsc

_NB = 5000
_C = 128
_FEAT = (256, 128, 64, 32)
_OUT_HW = 7
_NBINS = _OUT_HW * _OUT_HW
_NPTS = _NBINS * 16
_NCHUNK = 7
_CHUNK_ROWS = 112
_NBUF = 4
_NWORKERS = 32
_PER_TILE = 157
_NB_PAD = _NWORKERS * _PER_TILE
_TABLE_ROWS = sum(s * s for s in _FEAT)
_CANONICAL_SCALE = 224.0
_CANONICAL_LEVEL = 4.0


def _splat(ref, pos):
    return plsc.load_gather(ref, [jnp.full((16,), pos, jnp.int32)])


_HB = 8


def _make_transpose_body(w):
    def body(s_ref, *rest):
        o_ref = rest[-1]
        for r in range(_HB):
            o_ref[pl.ds(r * w, w), :] = jnp.transpose(
                s_ref[0, :, r, :], (1, 0)).astype(jnp.bfloat16)
    return body


def _build_table(feats):
    table = None
    off = 0
    for f in feats:
        h, w = f.shape[2], f.shape[3]
        rows_per_blk = _HB * w
        off_blk = off // rows_per_blk
        out_shape = jax.ShapeDtypeStruct((_TABLE_ROWS, _C), jnp.bfloat16)
        out_spec = pl.BlockSpec(
            (rows_per_blk, _C), lambda g, o=off_blk: (o + g, 0))
        in_spec = pl.BlockSpec((1, _C, _HB, w), lambda g: (0, 0, g, 0))
        body = _make_transpose_body(w)
        if table is None:
            table = pl.pallas_call(
                body,
                grid=(h // _HB,),
                in_specs=[in_spec],
                out_specs=out_spec,
                out_shape=out_shape,
            )(f)
        else:
            table = pl.pallas_call(
                body,
                grid=(h // _HB,),
                in_specs=[in_spec,
                          pl.BlockSpec(memory_space=pl.ANY)],
                out_specs=out_spec,
                out_shape=out_shape,
                input_output_aliases={1: 0},
            )(f, table)
        off += h * w
    return table


def _roi_body(table, bparams, lvlp, out, bp_v, lvlp_v, idx_v, w_v, rows_v,
              acc_v, sem_stage, g0, g1, g2, g3):
    gsems = (g0, g1, g2, g3)
    cid = lax.axis_index("c")
    sid = lax.axis_index("s")
    wid = sid * 2 + cid
    base = wid * _PER_TILE
    nb = jnp.minimum(_PER_TILE, _NB - base)

    pltpu.async_copy(bparams.at[pl.ds(base * 8, _PER_TILE * 8)], bp_v,
                     sem_stage).wait()
    pltpu.async_copy(lvlp, lvlp_v, sem_stage).wait()

    lane = lax.iota(jnp.int32, 16)
    syf = ((lane >> 3) & 1).astype(jnp.float32) + 0.5
    sxf = ((lane >> 2) & 1).astype(jnp.float32) + 0.5
    dym = ((lane >> 1) & 1) == 1
    dxm = (lane & 1) == 1
    lane98 = (lane * 2) * _NBINS

    def per_box(i, _):
        b8 = i * 8
        x1 = _splat(bp_v, b8 + 0)
        y1 = _splat(bp_v, b8 + 1)
        x2 = _splat(bp_v, b8 + 2)
        y2 = _splat(bp_v, b8 + 3)
        li = _splat(bp_v, b8 + 4).astype(jnp.int32) * 4
        scale = plsc.load_gather(lvlp_v, [li])
        wf = plsc.load_gather(lvlp_v, [li + 1])
        hf = plsc.load_gather(lvlp_v, [li + 2])
        off_i = plsc.load_gather(lvlp_v, [li + 3]).astype(jnp.int32)
        rx1 = x1 * scale
        ry1 = y1 * scale
        roi_w = jnp.maximum(x2 * scale - rx1, 1.0)
        roi_h = jnp.maximum(y2 * scale - ry1, 1.0)
        bin_w = roi_w / 7.0
        bin_h = roi_h / 7.0
        bin_w2 = bin_w * 0.5
        bin_h2 = bin_h * 0.5
        wm1 = wf - 1.0
        hm1 = hf - 1.0
        w_i = wf.astype(jnp.int32)
        wm1_i = wm1.astype(jnp.int32)
        hm1_i = hm1.astype(jnp.int32)

        def per_vec(v, _):
            oh = v // _OUT_HW
            ow = v - oh * _OUT_HW
            ys = (ry1 + oh.astype(jnp.float32) * bin_h) + syf * bin_h2
            xs = (rx1 + ow.astype(jnp.float32) * bin_w) + sxf * bin_w2
            valid = ((ys >= -1.0) & (ys <= hf)) & ((xs >= -1.0) & (xs <= wf))
            ycl = jnp.minimum(jnp.maximum(ys, 0.0), hm1)
            xcl = jnp.minimum(jnp.maximum(xs, 0.0), wm1)
            y0i = ycl.astype(jnp.int32)
            x0i = xcl.astype(jnp.int32)
            ly = ycl - y0i.astype(jnp.float32)
            lx = xcl - x0i.astype(jnp.float32)
            wy = jnp.where(dym, ly, 1.0 - ly)
            wx = jnp.where(dxm, lx, 1.0 - lx)
            yc = jnp.where(dym, jnp.minimum(y0i + 1, hm1_i), y0i)
            xc = jnp.where(dxm, jnp.minimum(x0i + 1, wm1_i), x0i)
            w = (wy * wx) * jnp.where(valid, 0.25, 0.0)
            idx = (off_i + yc * w_i) + xc
            r = v // _OUT_HW
            cc = (v - r * _OUT_HW) * 16
            idx_v[r, pl.ds(cc, 16)] = idx
            w_v[pl.ds(v * 16, 16)] = w
            return ()

        lax.fori_loop(0, _NBINS, per_vec, (), unroll=False)

        handles = [None] * _NCHUNK

        def fire(j):
            slot = j % _NBUF
            handles[j] = pltpu.async_copy(
                table.at[idx_v.at[j]], rows_v.at[slot], gsems[slot])

        for j in range(_NBUF):
            fire(j)
        for j in range(_NCHUNK):
            slot = j % _NBUF
            handles[j].wait()

            def per_bin(bb, _):
                binno = j * _OUT_HW + bb
                acc_e = [None] * 4
                acc_o = [None] * 4
                rbase = bb * 16
                for g in range(4):
                    bacc = [None] * 4
                    for lg in range(4):
                        l = g * 4 + lg
                        wv = _splat(w_v, binno * 16 + l)
                        wb = plsc.pack(wv, wv,
                                       format=plsc.PackFormat.INTERLEAVED)
                        for c in range(4):
                            row = rows_v[slot, rbase + l, pl.ds(c * 32, 32)]
                            t = row * wb
                            bacc[c] = t if lg == 0 else bacc[c] + t
                    for c in range(4):
                        ev, ov = plsc.unpack(
                            bacc[c], format=plsc.PackFormat.INTERLEAVED,
                            preferred_element_type=jnp.float32)
                        if g == 0:
                            acc_e[c] = ev
                            acc_o[c] = ov
                        else:
                            acc_e[c] = acc_e[c] + ev
                            acc_o[c] = acc_o[c] + ov
                for c in range(4):
                    addr = lane98 + (c * 32 * _NBINS + binno)
                    plsc.store_scatter(acc_v, [addr], acc_e[c])
                    plsc.store_scatter(acc_v, [addr + _NBINS], acc_o[c])
                return ()

            lax.fori_loop(0, _OUT_HW, per_bin, (), unroll=False)
            if j + _NBUF < _NCHUNK:
                fire(j + _NBUF)

        pltpu.sync_copy(acc_v, out.at[base + i])
        return ()

    lax.fori_loop(0, nb, per_box, (), unroll=False)


@jax.jit
def _roialign_sc(table, bparams, lvlp):
    mesh = plsc.VectorSubcoreMesh(core_axis_name="c", subcore_axis_name="s")
    f = functools.partial(
        pl.kernel,
        out_type=jax.ShapeDtypeStruct((_NB, _C * _NBINS), jnp.float32),
        mesh=mesh,
        scratch_types=[
            pltpu.VMEM((_PER_TILE * 8,), jnp.float32),
            pltpu.VMEM((16,), jnp.float32),
            pltpu.VMEM((_NCHUNK, _CHUNK_ROWS), jnp.int32),
            pltpu.VMEM((_NPTS,), jnp.float32),
            pltpu.VMEM((_NBUF, _CHUNK_ROWS, _C), jnp.bfloat16),
            pltpu.VMEM((_C * _NBINS,), jnp.float32),
            pltpu.SemaphoreType.DMA,
            pltpu.SemaphoreType.DMA,
            pltpu.SemaphoreType.DMA,
            pltpu.SemaphoreType.DMA,
            pltpu.SemaphoreType.DMA,
        ],
        compiler_params=pltpu.CompilerParams(needs_layout_passes=False, use_tc_tiling_on_sc=False),
    )(_roi_body)
    return f(table, bparams, lvlp)


def kernel(feat0, feat1, feat2, feat3, boxes, image_h, image_w):
    feats = [feat0[0], feat1[0], feat2[0], feat3[0]]
    img_w = jnp.asarray(image_w, jnp.float32)
    scales = [jnp.asarray(float(f.shape[-1]), jnp.float32) / img_w
              for f in feats]
    lvl_min = -jnp.log2(scales[0])
    lvl_max = -jnp.log2(scales[-1])
    w = boxes[:, 2] - boxes[:, 0]
    h = boxes[:, 3] - boxes[:, 1]
    s = jnp.sqrt(w * h)
    t = jnp.floor(_CANONICAL_LEVEL + jnp.log2(s / _CANONICAL_SCALE + 1e-6))
    levels = jnp.clip(t, lvl_min, lvl_max) - lvl_min

    table = _build_table([feat0, feat1, feat2, feat3])

    offsets = []
    acc = 0
    for sz in _FEAT:
        offsets.append(acc)
        acc += sz * sz
    lvlp = jnp.stack(
        [jnp.stack([scales[l],
                    jnp.asarray(float(_FEAT[l]), jnp.float32),
                    jnp.asarray(float(_FEAT[l]), jnp.float32),
                    jnp.asarray(float(offsets[l]), jnp.float32)])
         for l in range(4)]).reshape(16)

    bparams = jnp.zeros((_NB_PAD, 8), jnp.float32)
    bparams = bparams.at[:_NB, :4].set(boxes)
    bparams = bparams.at[:_NB, 4].set(levels)
    out = _roialign_sc(table, bparams.reshape(-1), lvlp)
    return out.reshape(_NB, _C, _OUT_HW, _OUT_HW)
